# Initial kernel scaffold; baseline (speedup 1.0000x reference)
import jax
import jax.numpy as jnp
from jax.experimental import pallas as pl


def kernel(x, W_router, w_gate, w_up, w_down):
    raise NotImplementedError("write your pallas kernel here")


# speedup vs baseline: 1.3898x; 1.3898x over previous
import jax
import jax.numpy as jnp
from jax.experimental import pallas as pl
from jax.experimental.pallas import tpu as pltpu

T = 2048
D = 2048
E = 16
K = 8
F = 768

_RT = 256
_TM = 512


def _router_body(x_ref, wr_ref, comb_ref):
    x = x_ref[...].astype(jnp.bfloat16)
    w = wr_ref[...].astype(jnp.bfloat16)
    logits = jax.lax.dot_general(
        x, w, (((1,), (0,)), ((), ())), preferred_element_type=jnp.float32)
    m = jnp.max(logits, axis=-1, keepdims=True)
    ex = jnp.exp(logits - m)
    probs = ex / jnp.sum(ex, axis=-1, keepdims=True)
    p = probs
    sel = jnp.zeros(p.shape, dtype=jnp.bool_)
    idx = jax.lax.broadcasted_iota(jnp.int32, p.shape, 1)
    for _ in range(K):
        mx = jnp.max(p, axis=-1, keepdims=True)
        ismx = p == mx
        first_idx = jnp.min(jnp.where(ismx, idx, E), axis=-1, keepdims=True)
        pick = idx == first_idx
        sel = sel | pick
        p = jnp.where(pick, -jnp.inf, p)
    w8 = jnp.where(sel, probs, 0.0)
    comb_ref[...] = w8 / jnp.sum(w8, axis=-1, keepdims=True)


def _ffn_body(xb_ref, wg_ref, wu_ref, wd_ref, comb_ref, out_ref):
    e = pl.program_id(1)
    x = xb_ref[...]
    g = jax.lax.dot_general(
        x, wg_ref[0], (((1,), (0,)), ((), ())), preferred_element_type=jnp.float32)
    u = jax.lax.dot_general(
        x, wu_ref[0], (((1,), (0,)), ((), ())), preferred_element_type=jnp.float32)
    h = (g * jax.lax.logistic(g)) * u
    y = jax.lax.dot_general(
        h.astype(jnp.bfloat16), wd_ref[0], (((1,), (0,)), ((), ())),
        preferred_element_type=jnp.float32)
    lane = jax.lax.broadcasted_iota(jnp.int32, (1, E), 1)
    c = jnp.sum(jnp.where(lane == e, comb_ref[...], 0.0), axis=1, keepdims=True)
    contrib = y * c

    @pl.when(e == 0)
    def _():
        out_ref[...] = contrib

    @pl.when(e != 0)
    def _():
        out_ref[...] += contrib


def kernel(x, W_router, w_gate, w_up, w_down):
    combine = pl.pallas_call(
        _router_body,
        grid=(T // _RT,),
        in_specs=[
            pl.BlockSpec((_RT, D), lambda i: (i, 0)),
            pl.BlockSpec((D, E), lambda i: (0, 0)),
        ],
        out_specs=pl.BlockSpec((_RT, E), lambda i: (i, 0)),
        out_shape=jax.ShapeDtypeStruct((T, E), jnp.float32),
    )(x, W_router)

    xb = x.astype(jnp.bfloat16)
    wgb = w_gate.astype(jnp.bfloat16)
    wub = w_up.astype(jnp.bfloat16)
    wdb = w_down.astype(jnp.bfloat16)

    out = pl.pallas_call(
        _ffn_body,
        grid=(T // _TM, E),
        in_specs=[
            pl.BlockSpec((_TM, D), lambda i, e: (i, 0)),
            pl.BlockSpec((1, D, F), lambda i, e: (e, 0, 0)),
            pl.BlockSpec((1, D, F), lambda i, e: (e, 0, 0)),
            pl.BlockSpec((1, F, D), lambda i, e: (e, 0, 0)),
            pl.BlockSpec((_TM, E), lambda i, e: (i, 0)),
        ],
        out_specs=pl.BlockSpec((_TM, D), lambda i, e: (i, 0)),
        out_shape=jax.ShapeDtypeStruct((T, D), jnp.float32),
    )(xb, wgb, wub, wdb, combine)
    return out


# speedup vs baseline: 1.4100x; 1.0145x over previous
import jax
import jax.numpy as jnp
from jax.experimental import pallas as pl
from jax.experimental.pallas import tpu as pltpu

T = 2048
D = 2048
E = 16
K = 8
F = 768

_RT = 256
_TM = 1024


def _router_body(x_ref, wr_ref, comb_ref):
    x = x_ref[...].astype(jnp.bfloat16)
    w = wr_ref[...].astype(jnp.bfloat16)
    logits = jax.lax.dot_general(
        x, w, (((1,), (0,)), ((), ())), preferred_element_type=jnp.float32)
    m = jnp.max(logits, axis=-1, keepdims=True)
    ex = jnp.exp(logits - m)
    probs = ex / jnp.sum(ex, axis=-1, keepdims=True)
    p = probs
    sel = jnp.zeros(p.shape, dtype=jnp.bool_)
    idx = jax.lax.broadcasted_iota(jnp.int32, p.shape, 1)
    for _ in range(K):
        mx = jnp.max(p, axis=-1, keepdims=True)
        ismx = p == mx
        first_idx = jnp.min(jnp.where(ismx, idx, E), axis=-1, keepdims=True)
        pick = idx == first_idx
        sel = sel | pick
        p = jnp.where(pick, -jnp.inf, p)
    w8 = jnp.where(sel, probs, 0.0)
    comb_ref[...] = w8 / jnp.sum(w8, axis=-1, keepdims=True)


def _ffn_body(xb_ref, wg_ref, wu_ref, wd_ref, comb_ref, out_ref):
    e = pl.program_id(1)
    x = xb_ref[...]
    g = jax.lax.dot_general(
        x, wg_ref[0], (((1,), (0,)), ((), ())), preferred_element_type=jnp.float32)
    u = jax.lax.dot_general(
        x, wu_ref[0], (((1,), (0,)), ((), ())), preferred_element_type=jnp.float32)
    h = (g * jax.lax.logistic(g)) * u
    y = jax.lax.dot_general(
        h.astype(jnp.bfloat16), wd_ref[0], (((1,), (0,)), ((), ())),
        preferred_element_type=jnp.float32)
    lane = jax.lax.broadcasted_iota(jnp.int32, (1, E), 1)
    c = jnp.sum(jnp.where(lane == e, comb_ref[...], 0.0), axis=1, keepdims=True)
    contrib = y * c

    @pl.when(e == 0)
    def _():
        out_ref[...] = contrib

    @pl.when(e != 0)
    def _():
        out_ref[...] += contrib


def kernel(x, W_router, w_gate, w_up, w_down):
    combine = pl.pallas_call(
        _router_body,
        grid=(T // _RT,),
        in_specs=[
            pl.BlockSpec((_RT, D), lambda i: (i, 0)),
            pl.BlockSpec((D, E), lambda i: (0, 0)),
        ],
        out_specs=pl.BlockSpec((_RT, E), lambda i: (i, 0)),
        out_shape=jax.ShapeDtypeStruct((T, E), jnp.float32),
    )(x, W_router)

    xb = x.astype(jnp.bfloat16)
    wgb = w_gate.astype(jnp.bfloat16)
    wub = w_up.astype(jnp.bfloat16)
    wdb = w_down.astype(jnp.bfloat16)

    out = pl.pallas_call(
        _ffn_body,
        grid=(T // _TM, E),
        in_specs=[
            pl.BlockSpec((_TM, D), lambda i, e: (i, 0)),
            pl.BlockSpec((1, D, F), lambda i, e: (e, 0, 0)),
            pl.BlockSpec((1, D, F), lambda i, e: (e, 0, 0)),
            pl.BlockSpec((1, F, D), lambda i, e: (e, 0, 0)),
            pl.BlockSpec((_TM, E), lambda i, e: (i, 0)),
        ],
        out_specs=pl.BlockSpec((_TM, D), lambda i, e: (i, 0)),
        out_shape=jax.ShapeDtypeStruct((T, D), jnp.float32),
    )(xb, wgb, wub, wdb, combine)
    return out
